# baseline (speedup 1.0000x reference)
import functools

import jax
import jax.numpy as jnp
from jax import lax
from jax.experimental import pallas as pl
from jax.experimental.pallas import tpu as pltpu
from jax.experimental.pallas import tpu_sc as plsc

_D = 64
_R = 16384
_S = 50
_NC = 2
_NS = 16
_NW = _NC * _NS
_G = _R // _NW
_TB = 64
_NB = 4

_mesh = plsc.VectorSubcoreMesh(core_axis_name="c", subcore_axis_name="s")


@functools.partial(
    pl.kernel,
    mesh=_mesh,
    out_type=jax.ShapeDtypeStruct((_R, _S, _D), jnp.float32),
    scratch_types=[
        pltpu.VMEM((_TB, _S), jnp.int32),
        pltpu.VMEM((_TB, _D), jnp.int32),
        pltpu.VMEM((_NB, _D, 2 * _D), jnp.float32),
        pltpu.VMEM((_NB, _D, _D), jnp.float32),
        pltpu.SemaphoreType.DMA((_NB,)),
        pltpu.SemaphoreType.DMA((_NB,)),
    ],
)
def _gather_all(hidx_hbm, par_hbm, packed_hbm, out_hbm,
                hidx_v, par_v, pk_v, wb_v, gsem, wsem):
    wid = lax.axis_index("s") * _NC + lax.axis_index("c")
    base = wid * _G

    def g_start(g, slot):
        pltpu.async_copy(
            packed_hbm.at[hidx_v.at[g % _TB]],
            pk_v.at[slot, pl.ds(0, _S)], gsem.at[slot])

    def g_wait(slot):
        pltpu.make_async_copy(
            packed_hbm.at[hidx_v.at[0]],
            pk_v.at[slot, pl.ds(0, _S)], gsem.at[slot]).wait()

    def w_start(g, slot):
        pltpu.async_copy(
            wb_v.at[slot, pl.ds(0, _S)], out_hbm.at[base + g], wsem.at[slot])

    def w_wait(slot):
        pltpu.make_async_copy(
            wb_v.at[slot, pl.ds(0, _S)], out_hbm.at[base],
            wsem.at[slot]).wait()

    _dn = lax.GatherDimensionNumbers(
        offset_dims=(), collapsed_slice_dims=(0,), start_index_map=(0,))

    def compact(br, slot):
        def grp(k, carry):
            o = pl.multiple_of(k * 16, 16)
            pf16 = par_v[br, pl.ds(o, 16)].astype(jnp.float32)
            for i in range(16):
                pf = lax.gather(
                    pf16, jnp.full((16, 1), i, jnp.int32), _dn, (1,),
                    mode=lax.GatherScatterMode.PROMISE_IN_BOUNDS)
                t = o + i
                for j in range(4):
                    a = pk_v[slot, t, pl.ds(16 * j, 16)]
                    b = pk_v[slot, t, pl.ds(_D + 16 * j, 16)]
                    wb_v[slot, t, pl.ds(16 * j, 16)] = a + (b - a) * pf
            return carry

        lax.fori_loop(0, _D // 16, grp, 0)

    def body(g4, carry):
        for p in range(_NB):
            g = _NB * g4 + p
            br = lax.rem(g, _TB)

            @pl.when(br == 0)
            def _():
                start = pl.multiple_of(base + g, _TB)
                pltpu.sync_copy(hidx_hbm.at[pl.ds(start, _TB)], hidx_v)
                pltpu.sync_copy(par_hbm.at[pl.ds(start, _TB)], par_v)
                for q in range(_NB - 1):
                    g_start(g + q, (p + q) % _NB)

            g_wait(p)

            @pl.when(br < _TB - (_NB - 1))
            def _():
                g_start(g + _NB - 1, (p + _NB - 1) % _NB)

            @pl.when(g >= _NB)
            def _():
                w_wait(p)

            compact(br, p)
            w_start(g, p)
        return carry

    lax.fori_loop(0, _G // _NB, body, 0)

    for p in range(_NB):
        w_wait(p)


def kernel(token_ids, embedding):
    ids = token_ids.astype(jnp.int32)
    hidx = lax.shift_right_logical(ids, 1)
    par = jnp.pad(lax.bitwise_and(ids, 1), ((0, 0), (0, _D - _S)))
    packed = jnp.reshape(embedding, (embedding.shape[0] // 2, 2 * _D))
    return _gather_all(hidx, par, packed)


# speedup vs baseline: 1.4146x; 1.4146x over previous
import functools

import jax
import jax.numpy as jnp
from jax import lax
from jax.experimental import pallas as pl
from jax.experimental.pallas import tpu as pltpu
from jax.experimental.pallas import tpu_sc as plsc

_D = 64
_R = 16384
_S = 50
_NC = 2
_NS = 16
_NW = _NC * _NS
_G = _R // _NW
_TB = 64
_NB = 8

_mesh = plsc.VectorSubcoreMesh(core_axis_name="c", subcore_axis_name="s")


@functools.partial(
    pl.kernel,
    mesh=_mesh,
    out_type=jax.ShapeDtypeStruct((_R, _S, _D), jnp.float32),
    compiler_params=pltpu.CompilerParams(use_tc_tiling_on_sc=False),
    scratch_types=[
        pltpu.VMEM((_TB, _S), jnp.int32),
        pltpu.VMEM((_NB, _S, _D), jnp.float32),
        pltpu.SemaphoreType.DMA((_NB,)),
        pltpu.SemaphoreType.DMA((_NB,)),
    ],
)
def _gather_all(ids_hbm, table_hbm, out_hbm, idx_v, rows_v, gsem, wsem):
    wid = lax.axis_index("s") * _NC + lax.axis_index("c")
    base = wid * _G

    def g_start(g, slot):
        pltpu.async_copy(
            table_hbm.at[idx_v.at[g % _TB]], rows_v.at[slot], gsem.at[slot])

    def g_wait(slot):
        pltpu.make_async_copy(
            table_hbm.at[idx_v.at[0]], rows_v.at[slot], gsem.at[slot]).wait()

    def w_start(g, slot):
        pltpu.async_copy(rows_v.at[slot], out_hbm.at[base + g], wsem.at[slot])

    def w_wait(slot):
        pltpu.make_async_copy(
            rows_v.at[slot], out_hbm.at[base], wsem.at[slot]).wait()

    def body(g4, carry):
        for p in range(_NB):
            g = _NB * g4 + p
            br = lax.rem(g, _TB)

            @pl.when(br == 0)
            def _():
                start = pl.multiple_of(base + g, _TB)
                pltpu.sync_copy(ids_hbm.at[pl.ds(start, _TB)], idx_v)
                for q in range(_NB - 1):
                    slot = (p + q) % _NB

                    @pl.when(g + q >= _NB)
                    def _():
                        w_wait(slot)

                    g_start(g + q, slot)

            g_wait(p)
            w_start(g, p)

            @pl.when(br < _TB - (_NB - 1))
            def _():
                nslot = (p + _NB - 1) % _NB

                @pl.when(g >= 1)
                def _():
                    w_wait(nslot)

                g_start(g + _NB - 1, nslot)
        return carry

    lax.fori_loop(0, _G // _NB, body, 0)

    for p in range(_NB):
        w_wait(p)


def kernel(token_ids, embedding):
    return _gather_all(token_ids.astype(jnp.int32), embedding)


# speedup vs baseline: 1.4215x; 1.0049x over previous
import functools

import jax
import jax.numpy as jnp
from jax import lax
from jax.experimental import pallas as pl
from jax.experimental.pallas import tpu as pltpu
from jax.experimental.pallas import tpu_sc as plsc

_D = 64
_R = 8192
_S = 100
_NC = 2
_NS = 16
_NW = _NC * _NS
_G = _R // _NW
_TB = 64
_NB = 8

_mesh = plsc.VectorSubcoreMesh(core_axis_name="c", subcore_axis_name="s")


@functools.partial(
    pl.kernel,
    mesh=_mesh,
    out_type=jax.ShapeDtypeStruct((_R, _S, _D), jnp.float32),
    compiler_params=pltpu.CompilerParams(use_tc_tiling_on_sc=False),
    scratch_types=[
        pltpu.VMEM((_TB, _S), jnp.int32),
        pltpu.VMEM((_NB, _S, _D), jnp.float32),
        pltpu.SemaphoreType.DMA((_NB,)),
        pltpu.SemaphoreType.DMA((_NB,)),
    ],
)
def _gather_all(ids_hbm, table_hbm, out_hbm, idx_v, rows_v, gsem, wsem):
    wid = lax.axis_index("s") * _NC + lax.axis_index("c")
    base = wid * _G

    def g_start(g, slot):
        pltpu.async_copy(
            table_hbm.at[idx_v.at[g % _TB]], rows_v.at[slot], gsem.at[slot])

    def g_wait(slot):
        pltpu.make_async_copy(
            table_hbm.at[idx_v.at[0]], rows_v.at[slot], gsem.at[slot]).wait()

    def w_start(g, slot):
        pltpu.async_copy(rows_v.at[slot], out_hbm.at[base + g], wsem.at[slot])

    def w_wait(slot):
        pltpu.make_async_copy(
            rows_v.at[slot], out_hbm.at[base], wsem.at[slot]).wait()

    def body(g4, carry):
        for p in range(_NB):
            g = _NB * g4 + p
            br = lax.rem(g, _TB)

            @pl.when(br == 0)
            def _():
                start = pl.multiple_of(base + g, _TB)
                pltpu.sync_copy(ids_hbm.at[pl.ds(start, _TB)], idx_v)
                for q in range(_NB - 1):
                    slot = (p + q) % _NB

                    @pl.when(g + q >= _NB)
                    def _():
                        w_wait(slot)

                    g_start(g + q, slot)

            g_wait(p)
            w_start(g, p)

            @pl.when(br < _TB - (_NB - 1))
            def _():
                nslot = (p + _NB - 1) % _NB

                @pl.when(g >= 1)
                def _():
                    w_wait(nslot)

                g_start(g + _NB - 1, nslot)
        return carry

    lax.fori_loop(0, _G // _NB, body, 0)

    for p in range(_NB):
        w_wait(p)


def kernel(token_ids, embedding):
    ids = jnp.reshape(token_ids.astype(jnp.int32), (_R, _S))
    out = _gather_all(ids, embedding)
    return jnp.reshape(out, (16384, 50, _D))
